# Initial kernel scaffold; baseline (speedup 1.0000x reference)
import jax
import jax.numpy as jnp
from jax.experimental import pallas as pl


def kernel(features, edge_index, W1, b1, W2, b2, W5, b5):
    raise NotImplementedError("write your pallas kernel here")


# speedup vs baseline: 7.2362x; 7.2362x over previous
import jax
import jax.numpy as jnp
from jax import lax
from jax.experimental import pallas as pl
from jax.experimental.pallas import tpu as pltpu
from jax.experimental.pallas import tpu_sc as plsc
from jax.experimental.layout import Format, Layout, with_layout_constraint


def _linear(x):
    sharding = jax.sharding.SingleDeviceSharding(jax.devices()[0])
    return jax.device_put(
        x, Format(Layout(major_to_minor=(0, 1), tiling=((16,),)), sharding))

N = 100000
E = 1600000
B = 125
KB = 8
NROWS = E // B
NSB = NROWS // KB
U = 200
NU = N // U
F32 = jnp.float32
I32 = jnp.int32


def _zero_fill(buf):
    def body(i, _):
        buf[i, :] = jnp.zeros((16,), F32)
        return 0
    lax.fori_loop(0, U, body, 0)


def _edge_pass(xflat, src_idx, dst_rows, worker, nworkers, acc,
               srcbuf, dstbuf, rows, sem_g, sem_s):
    def it(t, _):
        r0 = (worker + nworkers * t) * KB
        pltpu.sync_copy(src_idx.at[pl.ds(r0, KB)], srcbuf)
        pltpu.sync_copy(dst_rows.at[pl.ds(r0, KB)], dstbuf)
        for j in range(KB):
            pltpu.async_copy(xflat.at[srcbuf.at[j]], rows.at[j], sem_g)
        for j in range(KB):
            pltpu.make_async_copy(xflat.at[srcbuf.at[j]], rows.at[j], sem_g).wait()
        for j in range(KB):
            pltpu.async_copy(rows.at[j], acc.at[dstbuf.at[j]], sem_s, add=True)
        for j in range(KB):
            pltpu.make_async_copy(rows.at[j], acc.at[dstbuf.at[j]], sem_s).wait()
        return 0
    lax.fori_loop(0, NSB // nworkers, it, 0)


def _zero_acc(acc, zbuf, sub):
    for t in range(32):
        u = sub + 16 * t
        @pl.when(u < NU)
        def _():
            pltpu.sync_copy(zbuf, acc.at[pl.ds(u * U, U)])


def _writeback(acc, stage, out, out_base, sub):
    for t in range(32):
        u = sub + 16 * t
        @pl.when(u < NU)
        def _():
            pltpu.sync_copy(acc.at[pl.ds(u * U, U)], stage)
            pltpu.sync_copy(stage, out.at[pl.ds(out_base + u * U, U)])


def _agg_mid_body(xflat, srcall, dst2d, out,
                  acc, srcbuf, dstbuf, rows, zbuf, stage, sem_g, sem_s):
    core = lax.axis_index("c")
    sub = lax.axis_index("s")
    _zero_fill(zbuf)
    for l in range(2):
        cc = 2 * core + l
        _zero_acc(acc, zbuf, sub)
        plsc.subcore_barrier()
        _edge_pass(xflat, srcall.at[cc], dst2d, sub, 16,
                   acc, srcbuf, dstbuf, rows, sem_g, sem_s)
        plsc.subcore_barrier()
        _writeback(acc, stage, out, cc * N, sub)
        plsc.subcore_barrier()


def _agg_split_body(xtab, src2d, dst2d, out,
                    acc, srcbuf, dstbuf, rows, zbuf, stage, sem_g, sem_s):
    core = lax.axis_index("c")
    sub = lax.axis_index("s")
    _zero_fill(zbuf)
    wid = core * 16 + sub
    _zero_acc(acc, zbuf, sub)
    plsc.subcore_barrier()
    _edge_pass(xtab, src2d, dst2d, wid, 32,
               acc, srcbuf, dstbuf, rows, sem_g, sem_s)
    plsc.subcore_barrier()
    _writeback(acc, stage, out, core * N, sub)


_SC_SCRATCH = [
    pltpu.VMEM_SHARED((N, 16), F32),
    pltpu.VMEM((KB, B), I32),
    pltpu.VMEM((KB, B), I32),
    pltpu.VMEM((KB, B, 16), F32),
    pltpu.VMEM((U, 16), F32),
    pltpu.VMEM((U, 16), F32),
    pltpu.SemaphoreType.DMA,
    pltpu.SemaphoreType.DMA,
]
_SC_MESH = plsc.VectorSubcoreMesh(core_axis_name="c", subcore_axis_name="s")

_SC_PARAMS = pltpu.CompilerParams(use_tc_tiling_on_sc=False)

_agg_mid = pl.kernel(
    _agg_mid_body,
    out_type=jax.ShapeDtypeStruct((4 * N, 16), F32),
    mesh=_SC_MESH,
    scratch_types=_SC_SCRATCH,
    compiler_params=_SC_PARAMS,
    name="sc_agg_mid",
)

_agg_split = pl.kernel(
    _agg_split_body,
    out_type=jax.ShapeDtypeStruct((2 * N, 16), F32),
    mesh=_SC_MESH,
    scratch_types=_SC_SCRATCH,
    compiler_params=_SC_PARAMS,
    name="sc_agg_split",
)


R = 2000
_GRID = N // R


def _mm1_body(p_ref, w_ref, b_ref, o_ref):
    a = p_ref[0] + p_ref[1]
    acc = jnp.dot(a, w_ref[...], preferred_element_type=F32)
    acc += b_ref[0, :][None, :]
    y = jnp.where(acc >= 0, acc, 0.01 * acc)
    for c in range(4):
        o_ref[c] = y[:, 16 * c:16 * (c + 1)]


def _mm_mid_body(a_ref, w_ref, b_ref, o_ref):
    acc = jnp.dot(a_ref[0], w_ref[0:16, :], preferred_element_type=F32)
    for c in range(1, 4):
        acc += jnp.dot(a_ref[c], w_ref[16 * c:16 * (c + 1), :],
                       preferred_element_type=F32)
    acc += b_ref[0, :][None, :]
    y = jnp.where(acc >= 0, acc, 0.01 * acc)
    for c in range(4):
        o_ref[c] = y[:, 16 * c:16 * (c + 1)]


def _mm_last_body(a_ref, w_ref, o_ref):
    acc = jnp.dot(a_ref[0], w_ref[0:16, :], preferred_element_type=F32)
    for c in range(1, 4):
        acc += jnp.dot(a_ref[c], w_ref[16 * c:16 * (c + 1), :],
                       preferred_element_type=F32)
    o_ref[...] = acc


def _sum_bias_body(p_ref, b_ref, o_ref):
    o_ref[...] = p_ref[0] + p_ref[1] + b_ref[0, :][None, :]


def _full(shape):
    return pl.BlockSpec(shape, lambda i: tuple(0 for _ in shape))


_mm1 = pl.pallas_call(
    _mm1_body,
    grid=(_GRID,),
    in_specs=[pl.BlockSpec((2, R, 16), lambda i: (0, i, 0)),
              _full((16, 64)), _full((1, 64))],
    out_specs=pl.BlockSpec((4, R, 16), lambda i: (0, i, 0)),
    out_shape=jax.ShapeDtypeStruct((4, N, 16), F32),
)

_mm_mid = pl.pallas_call(
    _mm_mid_body,
    grid=(_GRID,),
    in_specs=[pl.BlockSpec((4, R, 16), lambda i: (0, i, 0)),
              _full((64, 64)), _full((1, 64))],
    out_specs=pl.BlockSpec((4, R, 16), lambda i: (0, i, 0)),
    out_shape=jax.ShapeDtypeStruct((4, N, 16), F32),
)

_mm_last = pl.pallas_call(
    _mm_last_body,
    grid=(_GRID,),
    in_specs=[pl.BlockSpec((4, R, 16), lambda i: (0, i, 0)),
              _full((64, 16))],
    out_specs=pl.BlockSpec((R, 16), lambda i: (i, 0)),
    out_shape=jax.ShapeDtypeStruct((N, 16), F32),
)

_sum_bias = pl.pallas_call(
    _sum_bias_body,
    grid=(_GRID,),
    in_specs=[pl.BlockSpec((2, R, 16), lambda i: (0, i, 0)),
              _full((1, 16))],
    out_specs=pl.BlockSpec((R, 16), lambda i: (i, 0)),
    out_shape=jax.ShapeDtypeStruct((N, 16), F32),
)


def kernel(features, edge_index, W1, b1, W2, b2, W5, b5):
    src = edge_index[0].astype(I32)
    dst = edge_index[1].astype(I32)
    src2d = src.reshape(NROWS, B)
    dst2d = dst.reshape(NROWS, B)
    srcall = (src[None, :] + (jnp.arange(4, dtype=I32) * N)[:, None]
              ).reshape(4, NROWS, B)

    feat16 = jnp.pad(features, ((0, 0), (0, 12)))
    W1p = jnp.pad(W1, ((0, 12), (0, 0)))
    W5p = jnp.pad(W5, ((0, 0), (0, 13)))
    b1r = b1.reshape(1, 64)
    b2r = b2.reshape(1, 64)
    b5p = jnp.pad(b5, (0, 13)).reshape(1, 16)

    p = _agg_split(_linear(feat16), src2d, dst2d).reshape(2, N, 16)
    x = _mm1(p, W1p, b1r)
    for _ in range(7):
        a = _agg_mid(_linear(x.reshape(4 * N, 16)), srcall, dst2d
                     ).reshape(4, N, 16)
        x = _mm_mid(a, W2, b2r)
    y = _mm_last(x, W5p)
    p = _agg_split(_linear(y), src2d, dst2d).reshape(2, N, 16)
    o16 = _sum_bias(p, b5p)
    return o16[:, :3]


# speedup vs baseline: 8.1904x; 1.1319x over previous
import jax
import jax.numpy as jnp
from jax import lax
from jax.experimental import pallas as pl
from jax.experimental.pallas import tpu as pltpu
from jax.experimental.pallas import tpu_sc as plsc
from jax.experimental.layout import Format, Layout, with_layout_constraint


def _linear(x):
    sharding = jax.sharding.SingleDeviceSharding(jax.devices()[0])
    return jax.device_put(
        x, Format(Layout(major_to_minor=(0, 1), tiling=((16,),)), sharding))

N = 100000
E = 1600000
B = 125
KB = 4
NROWS = E // B
NSB = NROWS // KB
U = 200
NU = N // U
F32 = jnp.float32
I32 = jnp.int32


def _zero_fill(buf):
    def body(i, _):
        buf[i, :] = jnp.zeros((16,), F32)
        return 0
    lax.fori_loop(0, U, body, 0)


def _edge_pass(xflat, sd, worker, nworkers, acc,
               sdbuf, rows, sem_g, sem_s):
    T = NSB // nworkers

    def fire_g(t, slot):
        pltpu.sync_copy(sd.at[worker + nworkers * t], sdbuf.at[slot])
        for j in range(KB):
            pltpu.async_copy(xflat.at[sdbuf.at[slot].at[j]],
                             rows.at[slot * KB + j], sem_g)

    def drain_g(slot):
        for j in range(KB):
            pltpu.make_async_copy(xflat.at[sdbuf.at[slot].at[j]],
                                  rows.at[slot * KB + j], sem_g).wait()

    def fire_s(slot):
        for j in range(KB):
            pltpu.async_copy(rows.at[slot * KB + j],
                             acc.at[sdbuf.at[slot].at[KB + j]], sem_s,
                             add=True)

    def drain_s(slot):
        for j in range(KB):
            pltpu.make_async_copy(rows.at[slot * KB + j],
                                  acc.at[sdbuf.at[slot].at[KB + j]],
                                  sem_s).wait()

    fire_g(0, 0)

    def body(t, _):
        slot = lax.rem(t, 2)
        nslot = 1 - slot

        @pl.when(t > 0)
        def _():
            drain_s(nslot)

        @pl.when(t + 1 < T)
        def _():
            fire_g(t + 1, nslot)

        drain_g(slot)
        fire_s(slot)
        return 0

    lax.fori_loop(0, T, body, 0)
    drain_s(lax.rem(T - 1, 2))


def _zero_acc(acc, zbuf, sub):
    for t in range(32):
        u = sub + 16 * t
        @pl.when(u < NU)
        def _():
            pltpu.sync_copy(zbuf, acc.at[pl.ds(u * U, U)])


def _writeback(acc, stage, out, out_base, sub):
    for t in range(32):
        u = sub + 16 * t
        @pl.when(u < NU)
        def _():
            pltpu.sync_copy(acc.at[pl.ds(u * U, U)], stage)
            pltpu.sync_copy(stage, out.at[pl.ds(out_base + u * U, U)])


def _agg_mid_body(xflat, sdall, out,
                  acc, sdbuf, rows, zbuf, stage, sem_g, sem_s):
    core = lax.axis_index("c")
    sub = lax.axis_index("s")
    _zero_fill(zbuf)
    for l in range(2):
        cc = 2 * core + l
        _zero_acc(acc, zbuf, sub)
        plsc.subcore_barrier()
        _edge_pass(xflat, sdall.at[cc], sub, 16,
                   acc, sdbuf, rows, sem_g, sem_s)
        plsc.subcore_barrier()
        _writeback(acc, stage, out, cc * N, sub)
        plsc.subcore_barrier()


def _agg_split_body(xtab, sd, out,
                    acc, sdbuf, rows, zbuf, stage, sem_g, sem_s):
    core = lax.axis_index("c")
    sub = lax.axis_index("s")
    _zero_fill(zbuf)
    wid = core * 16 + sub
    _zero_acc(acc, zbuf, sub)
    plsc.subcore_barrier()
    _edge_pass(xtab, sd, wid, 32,
               acc, sdbuf, rows, sem_g, sem_s)
    plsc.subcore_barrier()
    _writeback(acc, stage, out, core * N, sub)


_SC_SCRATCH = [
    pltpu.VMEM_SHARED((N, 16), F32),
    pltpu.VMEM((2, 2 * KB, B), I32),
    pltpu.VMEM((2 * KB, B, 16), F32),
    pltpu.VMEM((U, 16), F32),
    pltpu.VMEM((U, 16), F32),
    pltpu.SemaphoreType.DMA,
    pltpu.SemaphoreType.DMA,
]
_SC_MESH = plsc.VectorSubcoreMesh(core_axis_name="c", subcore_axis_name="s")

_SC_PARAMS = pltpu.CompilerParams(use_tc_tiling_on_sc=False)

_agg_mid = pl.kernel(
    _agg_mid_body,
    out_type=jax.ShapeDtypeStruct((4 * N, 16), F32),
    mesh=_SC_MESH,
    scratch_types=_SC_SCRATCH,
    compiler_params=_SC_PARAMS,
    name="sc_agg_mid",
)

_agg_split = pl.kernel(
    _agg_split_body,
    out_type=jax.ShapeDtypeStruct((2 * N, 16), F32),
    mesh=_SC_MESH,
    scratch_types=_SC_SCRATCH,
    compiler_params=_SC_PARAMS,
    name="sc_agg_split",
)


R = 2000
_GRID = N // R


def _mm1_body(p_ref, w_ref, b_ref, o_ref):
    a = p_ref[0] + p_ref[1]
    acc = jnp.dot(a, w_ref[...], preferred_element_type=F32)
    acc += b_ref[0, :][None, :]
    y = jnp.where(acc >= 0, acc, 0.01 * acc)
    for c in range(4):
        o_ref[c] = y[:, 16 * c:16 * (c + 1)]


def _mm_mid_body(a_ref, w_ref, b_ref, o_ref):
    acc = jnp.dot(a_ref[0], w_ref[0:16, :], preferred_element_type=F32)
    for c in range(1, 4):
        acc += jnp.dot(a_ref[c], w_ref[16 * c:16 * (c + 1), :],
                       preferred_element_type=F32)
    acc += b_ref[0, :][None, :]
    y = jnp.where(acc >= 0, acc, 0.01 * acc)
    for c in range(4):
        o_ref[c] = y[:, 16 * c:16 * (c + 1)]


def _mm_last_body(a_ref, w_ref, o_ref):
    acc = jnp.dot(a_ref[0], w_ref[0:16, :], preferred_element_type=F32)
    for c in range(1, 4):
        acc += jnp.dot(a_ref[c], w_ref[16 * c:16 * (c + 1), :],
                       preferred_element_type=F32)
    o_ref[...] = acc


def _sum_bias_body(p_ref, b_ref, o_ref):
    o_ref[...] = p_ref[0] + p_ref[1] + b_ref[0, :][None, :]


def _full(shape):
    return pl.BlockSpec(shape, lambda i: tuple(0 for _ in shape))


_mm1 = pl.pallas_call(
    _mm1_body,
    grid=(_GRID,),
    in_specs=[pl.BlockSpec((2, R, 16), lambda i: (0, i, 0)),
              _full((16, 64)), _full((1, 64))],
    out_specs=pl.BlockSpec((4, R, 16), lambda i: (0, i, 0)),
    out_shape=jax.ShapeDtypeStruct((4, N, 16), F32),
)

_mm_mid = pl.pallas_call(
    _mm_mid_body,
    grid=(_GRID,),
    in_specs=[pl.BlockSpec((4, R, 16), lambda i: (0, i, 0)),
              _full((64, 64)), _full((1, 64))],
    out_specs=pl.BlockSpec((4, R, 16), lambda i: (0, i, 0)),
    out_shape=jax.ShapeDtypeStruct((4, N, 16), F32),
)

_mm_last = pl.pallas_call(
    _mm_last_body,
    grid=(_GRID,),
    in_specs=[pl.BlockSpec((4, R, 16), lambda i: (0, i, 0)),
              _full((64, 16))],
    out_specs=pl.BlockSpec((R, 16), lambda i: (i, 0)),
    out_shape=jax.ShapeDtypeStruct((N, 16), F32),
)

_sum_bias = pl.pallas_call(
    _sum_bias_body,
    grid=(_GRID,),
    in_specs=[pl.BlockSpec((2, R, 16), lambda i: (0, i, 0)),
              _full((1, 16))],
    out_specs=pl.BlockSpec((R, 16), lambda i: (i, 0)),
    out_shape=jax.ShapeDtypeStruct((N, 16), F32),
)


def kernel(features, edge_index, W1, b1, W2, b2, W5, b5):
    src = edge_index[0].astype(I32)
    dst = edge_index[1].astype(I32)
    src3 = src.reshape(NSB, KB, B)
    dst3 = dst.reshape(NSB, KB, B)
    sd_split = jnp.concatenate([src3, dst3], axis=1)
    src4 = src3[None] + (jnp.arange(4, dtype=I32) * N)[:, None, None, None]
    dst4 = jnp.broadcast_to(dst3[None], (4, NSB, KB, B))
    sd_mid = jnp.concatenate([src4, dst4], axis=2)

    feat16 = jnp.pad(features, ((0, 0), (0, 12)))
    W1p = jnp.pad(W1, ((0, 12), (0, 0)))
    W5p = jnp.pad(W5, ((0, 0), (0, 13)))
    b1r = b1.reshape(1, 64)
    b2r = b2.reshape(1, 64)
    b5p = jnp.pad(b5, (0, 13)).reshape(1, 16)

    p = _agg_split(_linear(feat16), sd_split).reshape(2, N, 16)
    x = _mm1(p, W1p, b1r)
    for _ in range(7):
        a = _agg_mid(_linear(x.reshape(4 * N, 16)), sd_mid
                     ).reshape(4, N, 16)
        x = _mm_mid(a, W2, b2r)
    y = _mm_last(x, W5p)
    p = _agg_split(_linear(y), sd_split).reshape(2, N, 16)
    o16 = _sum_bias(p, b5p)
    return o16[:, :3]


# speedup vs baseline: 8.1937x; 1.0004x over previous
import jax
import jax.numpy as jnp
from jax import lax
from jax.experimental import pallas as pl
from jax.experimental.pallas import tpu as pltpu
from jax.experimental.pallas import tpu_sc as plsc
from jax.experimental.layout import Format, Layout, with_layout_constraint


def _linear(x):
    sharding = jax.sharding.SingleDeviceSharding(jax.devices()[0])
    return jax.device_put(
        x, Format(Layout(major_to_minor=(0, 1), tiling=((16,),)), sharding))

N = 100000
E = 1600000
B = 125
KB = 4
NROWS = E // B
NSB = NROWS // KB
U = 200
NU = N // U
F32 = jnp.float32
I32 = jnp.int32


def _zero_fill(buf):
    def body(i, _):
        buf[i, :] = jnp.zeros((16,), F32)
        return 0
    lax.fori_loop(0, U, body, 0)


def _edge_pass(xflat, sd, worker, nworkers, acc,
               sdbuf, rows, sem_g, sem_s):
    T = NSB // nworkers

    def fire_g(t, slot):
        pltpu.sync_copy(sd.at[worker + nworkers * t], sdbuf.at[slot])
        for j in range(KB):
            pltpu.async_copy(xflat.at[sdbuf.at[slot].at[j]],
                             rows.at[slot * KB + j], sem_g)

    def drain_g(slot):
        for j in range(KB):
            pltpu.make_async_copy(xflat.at[sdbuf.at[slot].at[j]],
                                  rows.at[slot * KB + j], sem_g).wait()

    def fire_s(slot):
        for j in range(KB):
            pltpu.async_copy(rows.at[slot * KB + j],
                             acc.at[sdbuf.at[slot].at[KB + j]], sem_s,
                             add=True)

    def drain_s(slot):
        for j in range(KB):
            pltpu.make_async_copy(rows.at[slot * KB + j],
                                  acc.at[sdbuf.at[slot].at[KB + j]],
                                  sem_s).wait()

    fire_g(0, 0)

    def body(t, _):
        slot = lax.rem(t, 2)
        nslot = 1 - slot

        @pl.when(t > 0)
        def _():
            drain_s(nslot)

        @pl.when(t + 1 < T)
        def _():
            fire_g(t + 1, nslot)

        drain_g(slot)
        fire_s(slot)
        return 0

    lax.fori_loop(0, T, body, 0)
    drain_s(lax.rem(T - 1, 2))


def _zero_acc(acc, zbuf, sub):
    for t in range(32):
        u = sub + 16 * t
        @pl.when(u < NU)
        def _():
            pltpu.sync_copy(zbuf, acc.at[pl.ds(u * U, U)])


def _writeback(acc, stage, out_plane, sub):
    for t in range(32):
        u = sub + 16 * t
        @pl.when(u < NU)
        def _():
            pltpu.sync_copy(acc.at[pl.ds(u * U, U)], stage)
            pltpu.sync_copy(stage, out_plane.at[pl.ds(u * U, U)])


def _agg_mid_body(xflat, sdall, out,
                  acc, sdbuf, rows, zbuf, stage, sem_g, sem_s):
    core = lax.axis_index("c")
    sub = lax.axis_index("s")
    _zero_fill(zbuf)
    for l in range(2):
        cc = 2 * core + l
        _zero_acc(acc, zbuf, sub)
        plsc.subcore_barrier()
        _edge_pass(xflat, sdall.at[cc], sub, 16,
                   acc, sdbuf, rows, sem_g, sem_s)
        plsc.subcore_barrier()
        _writeback(acc, stage, out.at[cc], sub)
        plsc.subcore_barrier()


def _agg_split_body(xtab, sd, out,
                    acc, sdbuf, rows, zbuf, stage, sem_g, sem_s):
    core = lax.axis_index("c")
    sub = lax.axis_index("s")
    _zero_fill(zbuf)
    wid = core * 16 + sub
    _zero_acc(acc, zbuf, sub)
    plsc.subcore_barrier()
    _edge_pass(xtab, sd, wid, 32,
               acc, sdbuf, rows, sem_g, sem_s)
    plsc.subcore_barrier()
    _writeback(acc, stage, out.at[core], sub)


_SC_SCRATCH = [
    pltpu.VMEM_SHARED((N, 16), F32),
    pltpu.VMEM((2, 2 * KB, B), I32),
    pltpu.VMEM((2 * KB, B, 16), F32),
    pltpu.VMEM((U, 16), F32),
    pltpu.VMEM((U, 16), F32),
    pltpu.SemaphoreType.DMA,
    pltpu.SemaphoreType.DMA,
]
_SC_MESH = plsc.VectorSubcoreMesh(core_axis_name="c", subcore_axis_name="s")

_SC_PARAMS = pltpu.CompilerParams(use_tc_tiling_on_sc=False)

_agg_mid = pl.kernel(
    _agg_mid_body,
    out_type=jax.ShapeDtypeStruct((4, N, 16), F32),
    mesh=_SC_MESH,
    scratch_types=_SC_SCRATCH,
    compiler_params=_SC_PARAMS,
    name="sc_agg_mid",
)

_agg_split = pl.kernel(
    _agg_split_body,
    out_type=jax.ShapeDtypeStruct((2, N, 16), F32),
    mesh=_SC_MESH,
    scratch_types=_SC_SCRATCH,
    compiler_params=_SC_PARAMS,
    name="sc_agg_split",
)


R = 2000
_GRID = N // R


def _mm1_body(p_ref, w_ref, b_ref, o_ref):
    a = p_ref[0] + p_ref[1]
    acc = jnp.dot(a, w_ref[...], preferred_element_type=F32)
    acc += b_ref[0, :][None, :]
    y = jnp.where(acc >= 0, acc, 0.01 * acc)
    for c in range(4):
        o_ref[c] = y[:, 16 * c:16 * (c + 1)]


def _mm_mid_body(a_ref, w_ref, b_ref, o_ref):
    acc = jnp.dot(a_ref[0], w_ref[0:16, :], preferred_element_type=F32)
    for c in range(1, 4):
        acc += jnp.dot(a_ref[c], w_ref[16 * c:16 * (c + 1), :],
                       preferred_element_type=F32)
    acc += b_ref[0, :][None, :]
    y = jnp.where(acc >= 0, acc, 0.01 * acc)
    for c in range(4):
        o_ref[c] = y[:, 16 * c:16 * (c + 1)]


def _mm_last_body(a_ref, w_ref, o_ref):
    acc = jnp.dot(a_ref[0], w_ref[0:16, :], preferred_element_type=F32)
    for c in range(1, 4):
        acc += jnp.dot(a_ref[c], w_ref[16 * c:16 * (c + 1), :],
                       preferred_element_type=F32)
    o_ref[...] = acc


def _sum_bias_body(p_ref, b_ref, o_ref):
    o_ref[...] = p_ref[0] + p_ref[1] + b_ref[0, :][None, :]


def _full(shape):
    return pl.BlockSpec(shape, lambda i: tuple(0 for _ in shape))


_mm1 = pl.pallas_call(
    _mm1_body,
    grid=(_GRID,),
    in_specs=[pl.BlockSpec((2, R, 16), lambda i: (0, i, 0)),
              _full((16, 64)), _full((1, 64))],
    out_specs=pl.BlockSpec((4, R, 16), lambda i: (0, i, 0)),
    out_shape=jax.ShapeDtypeStruct((4, N, 16), F32),
)

_mm_mid = pl.pallas_call(
    _mm_mid_body,
    grid=(_GRID,),
    in_specs=[pl.BlockSpec((4, R, 16), lambda i: (0, i, 0)),
              _full((64, 64)), _full((1, 64))],
    out_specs=pl.BlockSpec((4, R, 16), lambda i: (0, i, 0)),
    out_shape=jax.ShapeDtypeStruct((4, N, 16), F32),
)

_mm_last = pl.pallas_call(
    _mm_last_body,
    grid=(_GRID,),
    in_specs=[pl.BlockSpec((4, R, 16), lambda i: (0, i, 0)),
              _full((64, 16))],
    out_specs=pl.BlockSpec((R, 16), lambda i: (i, 0)),
    out_shape=jax.ShapeDtypeStruct((N, 16), F32),
)

_sum_bias = pl.pallas_call(
    _sum_bias_body,
    grid=(_GRID,),
    in_specs=[pl.BlockSpec((2, R, 16), lambda i: (0, i, 0)),
              _full((1, 16))],
    out_specs=pl.BlockSpec((R, 16), lambda i: (i, 0)),
    out_shape=jax.ShapeDtypeStruct((N, 16), F32),
)


def kernel(features, edge_index, W1, b1, W2, b2, W5, b5):
    src = edge_index[0].astype(I32)
    dst = edge_index[1].astype(I32)
    src3 = src.reshape(NSB, KB, B)
    dst3 = dst.reshape(NSB, KB, B)
    sd_split = jnp.concatenate([src3, dst3], axis=1)
    src4 = src3[None] + (jnp.arange(4, dtype=I32) * N)[:, None, None, None]
    dst4 = jnp.broadcast_to(dst3[None], (4, NSB, KB, B))
    sd_mid = jnp.concatenate([src4, dst4], axis=2)

    feat16 = jnp.pad(features, ((0, 0), (0, 12)))
    W1p = jnp.pad(W1, ((0, 12), (0, 0)))
    W5p = jnp.pad(W5, ((0, 0), (0, 13)))
    b1r = b1.reshape(1, 64)
    b2r = b2.reshape(1, 64)
    b5p = jnp.pad(b5, (0, 13)).reshape(1, 16)

    p = _agg_split(_linear(feat16), sd_split)
    x = _mm1(p, W1p, b1r)
    for _ in range(7):
        a = _agg_mid(_linear(x.reshape(4 * N, 16)), sd_mid)
        x = _mm_mid(a, W2, b2r)
    y = _mm_last(x, W5p)
    p = _agg_split(_linear(y), sd_split)
    o16 = _sum_bias(p, b5p)
    return o16[:, :3]


# speedup vs baseline: 12.1018x; 1.4770x over previous
import jax
import jax.numpy as jnp
from jax import lax
from jax.experimental import pallas as pl
from jax.experimental.pallas import tpu as pltpu
from jax.experimental.pallas import tpu_sc as plsc
from jax.experimental.layout import Format, Layout, with_layout_constraint

N = 100000
NPAD = 102400
PR = NPAD // 8
E = 1600000
B = 125
KB = 4
NROWS = E // B
NSB = NROWS // KB
U = 64
NU = NPAD // U
U0 = 256
NU0 = NPAD // U0
F32 = jnp.float32
I32 = jnp.int32

_T16 = Layout(major_to_minor=(0, 1), tiling=((16,),))


def _as_table(x_packed, n_rows):
    t = x_packed.reshape(n_rows, 16)
    return with_layout_constraint(t, _T16)


def _linear_put(x):
    sharding = jax.sharding.SingleDeviceSharding(jax.devices()[0])
    return jax.device_put(x, Format(_T16, sharding))


def _zero_fill(buf, nrows):
    def body(i, _):
        buf[i, :] = jnp.zeros((16,), F32)
        return 0
    lax.fori_loop(0, nrows, body, 0)


def _edge_pass(xflat, sd, worker, nworkers, acc, sdbuf, rows, sem_g, sem_s):
    T = NSB // nworkers

    def fire_g(t, slot):
        pltpu.sync_copy(sd.at[worker + nworkers * t], sdbuf.at[slot])
        for j in range(KB):
            pltpu.async_copy(xflat.at[sdbuf.at[slot].at[j]],
                             rows.at[slot * KB + j], sem_g)

    def drain_g(slot):
        for j in range(KB):
            pltpu.make_async_copy(xflat.at[sdbuf.at[slot].at[j]],
                                  rows.at[slot * KB + j], sem_g).wait()

    def fire_s(slot):
        for j in range(KB):
            pltpu.async_copy(rows.at[slot * KB + j],
                             acc.at[sdbuf.at[slot].at[KB + j]], sem_s,
                             add=True)

    def drain_s(slot):
        for j in range(KB):
            pltpu.make_async_copy(rows.at[slot * KB + j],
                                  acc.at[sdbuf.at[slot].at[KB + j]],
                                  sem_s).wait()

    fire_g(0, 0)

    def body(t, _):
        slot = lax.rem(t, 2)
        nslot = 1 - slot

        @pl.when(t > 0)
        def _():
            drain_s(nslot)

        @pl.when(t + 1 < T)
        def _():
            fire_g(t + 1, nslot)

        drain_g(slot)
        fire_s(slot)
        return 0

    lax.fori_loop(0, T, body, 0)
    drain_s(lax.rem(T - 1, 2))


def _zero_acc(acc, zbuf, sub):
    def body(t, _):
        u = sub + 16 * t
        pltpu.sync_copy(zbuf, acc.at[pl.ds(u * U0, U0)])
        return 0
    lax.fori_loop(0, NU0 // 16, body, 0)


def _writeback(acc, stage, stage2, out_plane, sub):
    def body(t, _):
        u = sub + 16 * t
        pltpu.sync_copy(acc.at[pl.ds(u * U, U)], stage)
        for r8 in range(U // 8):
            for s in range(8):
                stage2[r8, pl.ds(16 * s, 16)] = stage[r8 * 8 + s, :]
        row0 = pl.multiple_of(u * (U // 8), 8)
        pltpu.sync_copy(stage2, out_plane.at[pl.ds(row0, U // 8)])
        return 0
    lax.fori_loop(0, NU // 16, body, 0)


def _agg_mid_body(xflat, sdall, out,
                  acc, sdbuf, rows, zbuf, stage, stage2, sem_g, sem_s):
    core = lax.axis_index("c")
    sub = lax.axis_index("s")
    _zero_fill(zbuf, U0)
    for l in range(2):
        cc = 2 * core + l
        _zero_acc(acc, zbuf, sub)
        plsc.subcore_barrier()
        _edge_pass(xflat, sdall.at[cc], sub, 16,
                   acc, sdbuf, rows, sem_g, sem_s)
        plsc.subcore_barrier()
        _writeback(acc, stage, stage2, out.at[cc], sub)
        plsc.subcore_barrier()


def _agg_split_body(xtab, sd, out,
                    acc, sdbuf, rows, zbuf, stage, stage2, sem_g, sem_s):
    core = lax.axis_index("c")
    sub = lax.axis_index("s")
    _zero_fill(zbuf, U0)
    wid = core * 16 + sub
    _zero_acc(acc, zbuf, sub)
    plsc.subcore_barrier()
    _edge_pass(xtab, sd, wid, 32, acc, sdbuf, rows, sem_g, sem_s)
    plsc.subcore_barrier()
    _writeback(acc, stage, stage2, out.at[core], sub)


_SC_SCRATCH = [
    pltpu.VMEM_SHARED((NPAD, 16), F32),
    pltpu.VMEM((2, 2 * KB, B), I32),
    pltpu.VMEM((2 * KB, B, 16), F32),
    pltpu.VMEM((U0, 16), F32),
    pltpu.VMEM((U, 16), F32),
    pltpu.VMEM((U // 8, 128), F32),
    pltpu.SemaphoreType.DMA,
    pltpu.SemaphoreType.DMA,
]
_SC_MESH = plsc.VectorSubcoreMesh(core_axis_name="c", subcore_axis_name="s")
_SC_PARAMS = pltpu.CompilerParams(use_tc_tiling_on_sc=False)

_agg_mid = pl.kernel(
    _agg_mid_body,
    out_type=jax.ShapeDtypeStruct((4, PR, 128), F32),
    mesh=_SC_MESH,
    scratch_types=_SC_SCRATCH,
    compiler_params=_SC_PARAMS,
    name="sc_agg_mid",
)

_agg_split = pl.kernel(
    _agg_split_body,
    out_type=jax.ShapeDtypeStruct((2, PR, 128), F32),
    mesh=_SC_MESH,
    scratch_types=_SC_SCRATCH,
    compiler_params=_SC_PARAMS,
    name="sc_agg_split",
)


BLK = 1600
_GRID = PR // BLK


def _leaky(x):
    return jnp.where(x >= 0, x, 0.01 * x)


def _mm1_body(p_ref, w_ref, b_ref, o_ref):
    a = p_ref[0] + p_ref[1]
    for c in range(4):
        acc = jnp.dot(a, w_ref[c], preferred_element_type=F32)
        o_ref[c] = _leaky(acc + b_ref[c, :][None, :])


def _mm_mid_body(a_ref, w_ref, b_ref, o_ref):
    for c in range(4):
        acc = jnp.dot(a_ref[0], w_ref[0, c], preferred_element_type=F32)
        for kc in range(1, 4):
            acc += jnp.dot(a_ref[kc], w_ref[kc, c],
                           preferred_element_type=F32)
        o_ref[c] = _leaky(acc + b_ref[c, :][None, :])


def _mm_last_body(a_ref, w_ref, o_ref):
    acc = jnp.dot(a_ref[0], w_ref[0], preferred_element_type=F32)
    for kc in range(1, 4):
        acc += jnp.dot(a_ref[kc], w_ref[kc], preferred_element_type=F32)
    o_ref[...] = acc


def _sum_bias_body(p_ref, b_ref, o_ref):
    o_ref[...] = p_ref[0] + p_ref[1] + b_ref[0, :][None, :]


def _full(shape):
    return pl.BlockSpec(shape, lambda i: tuple(0 for _ in shape))


_mm1 = pl.pallas_call(
    _mm1_body,
    grid=(_GRID,),
    in_specs=[pl.BlockSpec((2, BLK, 128), lambda i: (0, i, 0)),
              _full((4, 128, 128)), _full((4, 128))],
    out_specs=pl.BlockSpec((4, BLK, 128), lambda i: (0, i, 0)),
    out_shape=jax.ShapeDtypeStruct((4, PR, 128), F32),
)

_mm_mid = pl.pallas_call(
    _mm_mid_body,
    grid=(_GRID,),
    in_specs=[pl.BlockSpec((4, BLK, 128), lambda i: (0, i, 0)),
              _full((4, 4, 128, 128)), _full((4, 128))],
    out_specs=pl.BlockSpec((4, BLK, 128), lambda i: (0, i, 0)),
    out_shape=jax.ShapeDtypeStruct((4, PR, 128), F32),
)

_mm_last = pl.pallas_call(
    _mm_last_body,
    grid=(_GRID,),
    in_specs=[pl.BlockSpec((4, BLK, 128), lambda i: (0, i, 0)),
              _full((4, 128, 128))],
    out_specs=pl.BlockSpec((BLK, 128), lambda i: (i, 0)),
    out_shape=jax.ShapeDtypeStruct((PR, 128), F32),
)

_sum_bias = pl.pallas_call(
    _sum_bias_body,
    grid=(_GRID,),
    in_specs=[pl.BlockSpec((2, BLK, 128), lambda i: (0, i, 0)),
              _full((1, 128))],
    out_specs=pl.BlockSpec((BLK, 128), lambda i: (i, 0)),
    out_shape=jax.ShapeDtypeStruct((PR, 128), F32),
)


def _bd8(w16):
    w = jnp.zeros((16, 16), F32).at[:w16.shape[0], :w16.shape[1]].set(w16)
    return jnp.kron(jnp.eye(8, dtype=F32), w)


def kernel(features, edge_index, W1, b1, W2, b2, W5, b5):
    src = edge_index[0].astype(I32)
    dst = edge_index[1].astype(I32)
    src3 = src.reshape(NSB, KB, B)
    dst3 = dst.reshape(NSB, KB, B)
    sd_split = jnp.concatenate([src3, dst3], axis=1)
    src4 = src3[None] + (jnp.arange(4, dtype=I32) * NPAD)[:, None, None,
                                                          None]
    dst4 = jnp.broadcast_to(dst3[None], (4, NSB, KB, B))
    sd_mid = jnp.concatenate([src4, dst4], axis=2)

    feat16 = _linear_put(jnp.pad(features, ((0, 0), (0, 12))))

    bd1 = jnp.stack([_bd8(W1[:, 16 * c:16 * (c + 1)]) for c in range(4)])
    bd2 = jnp.stack(
        [jnp.stack([_bd8(W2[16 * kc:16 * (kc + 1), 16 * c:16 * (c + 1)])
                    for c in range(4)]) for kc in range(4)])
    bd5 = jnp.stack([_bd8(W5[16 * kc:16 * (kc + 1), :]) for kc in range(4)])
    b1p = jnp.stack([jnp.tile(b1[16 * c:16 * (c + 1)], 8) for c in range(4)])
    b2p = jnp.stack([jnp.tile(b2[16 * c:16 * (c + 1)], 8) for c in range(4)])
    b5p = jnp.tile(jnp.pad(b5, (0, 13)), 8).reshape(1, 128)

    p = _agg_split(feat16, sd_split)
    x = _mm1(p, bd1, b1p)
    for _ in range(7):
        a = _agg_mid(_as_table(x, 4 * NPAD), sd_mid)
        x = _mm_mid(a, bd2, b2p)
    y = _mm_last(x, bd5)
    p = _agg_split(_as_table(y, NPAD), sd_split)
    o = _sum_bias(p, b5p)
    return o.reshape(NPAD, 16)[:N, :3]


# speedup vs baseline: 12.8868x; 1.0649x over previous
import jax
import jax.numpy as jnp
from jax import lax
from jax.experimental import pallas as pl
from jax.experimental.pallas import tpu as pltpu
from jax.experimental.pallas import tpu_sc as plsc
from jax.experimental.layout import Format, Layout, with_layout_constraint

N = 100000
NPAD = 102400
PR = NPAD // 8
E = 1600000
B = 125
KB = 5
NROWS = E // B
NSB = NROWS // KB
U = 64
NU = NPAD // U
U0 = 128
NU0 = NPAD // U0
F32 = jnp.float32
I32 = jnp.int32

_T16 = Layout(major_to_minor=(0, 1), tiling=((16,),))


def _as_table(x_packed, n_rows):
    t = x_packed.reshape(n_rows, 16)
    return with_layout_constraint(t, _T16)


def _linear_put(x):
    sharding = jax.sharding.SingleDeviceSharding(jax.devices()[0])
    return jax.device_put(x, Format(_T16, sharding))


def _zero_fill(buf, nrows):
    def body(i, _):
        buf[i, :] = jnp.zeros((16,), F32)
        return 0
    lax.fori_loop(0, nrows, body, 0)


def _edge_pass(xflat, sd, worker, nworkers, acc, sdbuf, rows, sem_g, sem_s):
    T = NSB // nworkers

    def fire_g(t, slot):
        pltpu.sync_copy(sd.at[worker + nworkers * t], sdbuf.at[slot])
        for j in range(KB):
            pltpu.async_copy(xflat.at[sdbuf.at[slot].at[j]],
                             rows.at[slot * KB + j], sem_g)

    def drain_g(slot):
        for j in range(KB):
            pltpu.make_async_copy(xflat.at[sdbuf.at[slot].at[j]],
                                  rows.at[slot * KB + j], sem_g).wait()

    def fire_s(slot):
        for j in range(KB):
            pltpu.async_copy(rows.at[slot * KB + j],
                             acc.at[sdbuf.at[slot].at[KB + j]], sem_s,
                             add=True)

    def drain_s(slot):
        for j in range(KB):
            pltpu.make_async_copy(rows.at[slot * KB + j],
                                  acc.at[sdbuf.at[slot].at[KB + j]],
                                  sem_s).wait()

    fire_g(0, 0)

    def body(t, _):
        slot = lax.rem(t, 2)
        nslot = 1 - slot

        @pl.when(t > 0)
        def _():
            drain_s(nslot)

        @pl.when(t + 1 < T)
        def _():
            fire_g(t + 1, nslot)

        drain_g(slot)
        fire_s(slot)
        return 0

    lax.fori_loop(0, T, body, 0)
    drain_s(lax.rem(T - 1, 2))


def _zero_acc(acc, zbuf, sub):
    def body(t, _):
        u = sub + 16 * t
        pltpu.sync_copy(zbuf, acc.at[pl.ds(u * U0, U0)])
        return 0
    lax.fori_loop(0, NU0 // 16, body, 0)


def _writeback(acc, zbuf, stage, stage2, out_plane, sub):
    def body(t, _):
        u = sub + 16 * t
        pltpu.sync_copy(acc.at[pl.ds(u * U, U)], stage)
        pltpu.sync_copy(zbuf.at[pl.ds(0, U)], acc.at[pl.ds(u * U, U)])
        for r8 in range(U // 8):
            for s in range(8):
                stage2[r8, pl.ds(16 * s, 16)] = stage[r8 * 8 + s, :]
        row0 = pl.multiple_of(u * (U // 8), 8)
        pltpu.sync_copy(stage2, out_plane.at[pl.ds(row0, U // 8)])
        return 0
    lax.fori_loop(0, NU // 16, body, 0)


def _agg_mid_body(xflat, sdall, out,
                  acc, sdbuf, rows, zbuf, stage, stage2, sem_g, sem_s):
    core = lax.axis_index("c")
    sub = lax.axis_index("s")
    _zero_fill(zbuf, U0)
    _zero_acc(acc, zbuf, sub)
    for l in range(2):
        cc = 2 * core + l
        plsc.subcore_barrier()
        _edge_pass(xflat, sdall.at[cc], sub, 16,
                   acc, sdbuf, rows, sem_g, sem_s)
        plsc.subcore_barrier()
        _writeback(acc, zbuf, stage, stage2, out.at[cc], sub)


def _agg_split_body(xtab, sd, out,
                    acc, sdbuf, rows, zbuf, stage, stage2, sem_g, sem_s):
    core = lax.axis_index("c")
    sub = lax.axis_index("s")
    _zero_fill(zbuf, U0)
    wid = core * 16 + sub
    _zero_acc(acc, zbuf, sub)
    plsc.subcore_barrier()
    _edge_pass(xtab, sd, wid, 32, acc, sdbuf, rows, sem_g, sem_s)
    plsc.subcore_barrier()
    _writeback(acc, zbuf, stage, stage2, out.at[core], sub)


_SC_SCRATCH = [
    pltpu.VMEM_SHARED((NPAD, 16), F32),
    pltpu.VMEM((2, 2 * KB, B), I32),
    pltpu.VMEM((2 * KB, B, 16), F32),
    pltpu.VMEM((U0, 16), F32),
    pltpu.VMEM((U, 16), F32),
    pltpu.VMEM((U // 8, 128), F32),
    pltpu.SemaphoreType.DMA,
    pltpu.SemaphoreType.DMA,
]
_SC_MESH = plsc.VectorSubcoreMesh(core_axis_name="c", subcore_axis_name="s")
_SC_PARAMS = pltpu.CompilerParams(use_tc_tiling_on_sc=False)

_agg_mid = pl.kernel(
    _agg_mid_body,
    out_type=jax.ShapeDtypeStruct((4, PR, 128), F32),
    mesh=_SC_MESH,
    scratch_types=_SC_SCRATCH,
    compiler_params=_SC_PARAMS,
    name="sc_agg_mid",
)

_agg_split = pl.kernel(
    _agg_split_body,
    out_type=jax.ShapeDtypeStruct((2, PR, 128), F32),
    mesh=_SC_MESH,
    scratch_types=_SC_SCRATCH,
    compiler_params=_SC_PARAMS,
    name="sc_agg_split",
)


BLK = 1600
_GRID = PR // BLK


def _leaky(x):
    return jnp.where(x >= 0, x, 0.01 * x)


def _mm1_body(p_ref, w_ref, b_ref, o_ref):
    a = p_ref[0] + p_ref[1]
    for c in range(4):
        acc = jnp.dot(a, w_ref[c], preferred_element_type=F32)
        o_ref[c] = _leaky(acc + b_ref[c, :][None, :])


def _mm_mid_body(a_ref, w_ref, b_ref, o_ref):
    for c in range(4):
        acc = jnp.dot(a_ref[0], w_ref[0, c], preferred_element_type=F32)
        for kc in range(1, 4):
            acc += jnp.dot(a_ref[kc], w_ref[kc, c],
                           preferred_element_type=F32)
        o_ref[c] = _leaky(acc + b_ref[c, :][None, :])


def _mm_last_body(a_ref, w_ref, o_ref):
    acc = jnp.dot(a_ref[0], w_ref[0], preferred_element_type=F32)
    for kc in range(1, 4):
        acc += jnp.dot(a_ref[kc], w_ref[kc], preferred_element_type=F32)
    o_ref[...] = acc


def _sum_bias_body(p_ref, b_ref, o_ref):
    o_ref[...] = p_ref[0] + p_ref[1] + b_ref[0, :][None, :]


def _full(shape):
    return pl.BlockSpec(shape, lambda i: tuple(0 for _ in shape))


_mm1 = pl.pallas_call(
    _mm1_body,
    grid=(_GRID,),
    in_specs=[pl.BlockSpec((2, BLK, 128), lambda i: (0, i, 0)),
              _full((4, 128, 128)), _full((4, 128))],
    out_specs=pl.BlockSpec((4, BLK, 128), lambda i: (0, i, 0)),
    out_shape=jax.ShapeDtypeStruct((4, PR, 128), F32),
)

_mm_mid = pl.pallas_call(
    _mm_mid_body,
    grid=(_GRID,),
    in_specs=[pl.BlockSpec((4, BLK, 128), lambda i: (0, i, 0)),
              _full((4, 4, 128, 128)), _full((4, 128))],
    out_specs=pl.BlockSpec((4, BLK, 128), lambda i: (0, i, 0)),
    out_shape=jax.ShapeDtypeStruct((4, PR, 128), F32),
)

_mm_last = pl.pallas_call(
    _mm_last_body,
    grid=(_GRID,),
    in_specs=[pl.BlockSpec((4, BLK, 128), lambda i: (0, i, 0)),
              _full((4, 128, 128))],
    out_specs=pl.BlockSpec((BLK, 128), lambda i: (i, 0)),
    out_shape=jax.ShapeDtypeStruct((PR, 128), F32),
)

_sum_bias = pl.pallas_call(
    _sum_bias_body,
    grid=(_GRID,),
    in_specs=[pl.BlockSpec((2, BLK, 128), lambda i: (0, i, 0)),
              _full((1, 128))],
    out_specs=pl.BlockSpec((BLK, 128), lambda i: (i, 0)),
    out_shape=jax.ShapeDtypeStruct((PR, 128), F32),
)


def _bd8(w16):
    w = jnp.zeros((16, 16), F32).at[:w16.shape[0], :w16.shape[1]].set(w16)
    return jnp.kron(jnp.eye(8, dtype=F32), w)


def kernel(features, edge_index, W1, b1, W2, b2, W5, b5):
    src = edge_index[0].astype(I32)
    dst = edge_index[1].astype(I32)
    src3 = src.reshape(NSB, KB, B)
    dst3 = dst.reshape(NSB, KB, B)
    sd_split = jnp.concatenate([src3, dst3], axis=1)
    src4 = src3[None] + (jnp.arange(4, dtype=I32) * NPAD)[:, None, None,
                                                          None]
    dst4 = jnp.broadcast_to(dst3[None], (4, NSB, KB, B))
    sd_mid = jnp.concatenate([src4, dst4], axis=2)

    feat16 = _linear_put(jnp.pad(features, ((0, 0), (0, 12))))

    bd1 = jnp.stack([_bd8(W1[:, 16 * c:16 * (c + 1)]) for c in range(4)])
    bd2 = jnp.stack(
        [jnp.stack([_bd8(W2[16 * kc:16 * (kc + 1), 16 * c:16 * (c + 1)])
                    for c in range(4)]) for kc in range(4)])
    bd5 = jnp.stack([_bd8(W5[16 * kc:16 * (kc + 1), :]) for kc in range(4)])
    b1p = jnp.stack([jnp.tile(b1[16 * c:16 * (c + 1)], 8) for c in range(4)])
    b2p = jnp.stack([jnp.tile(b2[16 * c:16 * (c + 1)], 8) for c in range(4)])
    b5p = jnp.tile(jnp.pad(b5, (0, 13)), 8).reshape(1, 128)

    p = _agg_split(feat16, sd_split)
    x = _mm1(p, bd1, b1p)
    for _ in range(7):
        a = _agg_mid(_as_table(x, 4 * NPAD), sd_mid)
        x = _mm_mid(a, bd2, b2p)
    y = _mm_last(x, bd5)
    p = _agg_split(_as_table(y, NPAD), sd_split)
    o = _sum_bias(p, b5p)
    return o.reshape(NPAD, 16)[:N, :3]


# speedup vs baseline: 15.1414x; 1.1750x over previous
import jax
import jax.numpy as jnp
from jax import lax
from jax.experimental import pallas as pl
from jax.experimental.pallas import tpu as pltpu
from jax.experimental.pallas import tpu_sc as plsc
from jax.experimental.layout import Format, Layout, with_layout_constraint

N = 100000
NPAD = 100352
PR = NPAD // 8
E = 1600000
B = 125
KB = 4
NROWS = E // B
NSB = NROWS // KB
U = 64
NU = NPAD // U
U0 = 64
NU0 = NPAD // U0
F32 = jnp.float32
I32 = jnp.int32

_T16 = Layout(major_to_minor=(0, 1), tiling=((16,),))


def _as_table(x_packed, n_rows):
    t = x_packed.reshape(n_rows, 16)
    return with_layout_constraint(t, _T16)


def _linear_put(x):
    sharding = jax.sharding.SingleDeviceSharding(jax.devices()[0])
    return jax.device_put(x, Format(_T16, sharding))


def _zero_fill(buf, nrows):
    def body(i, _):
        buf[i, :] = jnp.zeros((16,), F32)
        return 0
    lax.fori_loop(0, nrows, body, 0)


def _edge_pass(xflat, sd, worker, nworkers, acc, sdbuf, rows, sem_g, sem_s):
    T = NSB // nworkers

    def fire_g(t, slot):
        pltpu.sync_copy(sd.at[worker + nworkers * t], sdbuf.at[slot])
        for j in range(KB):
            pltpu.async_copy(xflat.at[sdbuf.at[slot].at[j]],
                             rows.at[slot * KB + j], sem_g)

    def drain_g(slot):
        for j in range(KB):
            pltpu.make_async_copy(xflat.at[sdbuf.at[slot].at[j]],
                                  rows.at[slot * KB + j], sem_g).wait()

    def fire_s(slot):
        for j in range(KB):
            pltpu.async_copy(rows.at[slot * KB + j],
                             acc.at[sdbuf.at[slot].at[KB + j]], sem_s,
                             add=True)

    def drain_s(slot):
        for j in range(KB):
            pltpu.make_async_copy(rows.at[slot * KB + j],
                                  acc.at[sdbuf.at[slot].at[KB + j]],
                                  sem_s).wait()

    fire_g(0, 0)
    fire_g(1, 1)

    def body(t, _):
        slot = lax.rem(t, 3)

        @pl.when(t > 0)
        def _():
            drain_s(lax.rem(t + 2, 3))

        @pl.when(t + 2 < T)
        def _():
            fire_g(t + 2, lax.rem(t + 2, 3))

        drain_g(slot)
        fire_s(slot)
        return 0

    lax.fori_loop(0, T, body, 0)
    drain_s(lax.rem(T - 1, 3))


def _zero_acc(acc, zbuf, sub):
    def body(t, _):
        u = sub + 16 * t
        pltpu.sync_copy(zbuf, acc.at[pl.ds(u * U0, U0)])
        return 0
    lax.fori_loop(0, NU0 // 16, body, 0)


def _writeback(acc, zbuf, stage, stage2, out_plane, sub):
    def body(t, _):
        u = sub + 16 * t
        pltpu.sync_copy(acc.at[pl.ds(u * U, U)], stage)
        pltpu.sync_copy(zbuf.at[pl.ds(0, U)], acc.at[pl.ds(u * U, U)])
        for r8 in range(U // 8):
            for s in range(8):
                stage2[r8, pl.ds(16 * s, 16)] = stage[r8 * 8 + s, :]
        row0 = pl.multiple_of(u * (U // 8), 8)
        pltpu.sync_copy(stage2, out_plane.at[pl.ds(row0, U // 8)])
        return 0
    lax.fori_loop(0, NU // 16, body, 0)


def _agg_mid_body(xflat, sdall, out,
                  acc, sdbuf, rows, zbuf, stage, stage2, sem_g, sem_s):
    core = lax.axis_index("c")
    sub = lax.axis_index("s")
    _zero_fill(zbuf, U0)
    _zero_acc(acc, zbuf, sub)
    for l in range(2):
        cc = 2 * core + l
        plsc.subcore_barrier()
        _edge_pass(xflat, sdall.at[cc], sub, 16,
                   acc, sdbuf, rows, sem_g, sem_s)
        plsc.subcore_barrier()
        _writeback(acc, zbuf, stage, stage2, out.at[cc], sub)


def _agg_split_body(xtab, sd, out,
                    acc, sdbuf, rows, zbuf, stage, stage2, sem_g, sem_s):
    core = lax.axis_index("c")
    sub = lax.axis_index("s")
    _zero_fill(zbuf, U0)
    wid = core * 16 + sub
    _zero_acc(acc, zbuf, sub)
    plsc.subcore_barrier()
    _edge_pass(xtab, sd, wid, 32, acc, sdbuf, rows, sem_g, sem_s)
    plsc.subcore_barrier()
    _writeback(acc, zbuf, stage, stage2, out.at[core], sub)


_SC_SCRATCH = [
    pltpu.VMEM_SHARED((NPAD, 16), F32),
    pltpu.VMEM((3, 2 * KB, B), I32),
    pltpu.VMEM((3 * KB, B, 16), F32),
    pltpu.VMEM((U0, 16), F32),
    pltpu.VMEM((U, 16), F32),
    pltpu.VMEM((U // 8, 128), F32),
    pltpu.SemaphoreType.DMA,
    pltpu.SemaphoreType.DMA,
]
_SC_MESH = plsc.VectorSubcoreMesh(core_axis_name="c", subcore_axis_name="s")
_SC_PARAMS = pltpu.CompilerParams(use_tc_tiling_on_sc=False)

_agg_mid = pl.kernel(
    _agg_mid_body,
    out_type=jax.ShapeDtypeStruct((4, PR, 128), F32),
    mesh=_SC_MESH,
    scratch_types=_SC_SCRATCH,
    compiler_params=_SC_PARAMS,
    name="sc_agg_mid",
)

_agg_split = pl.kernel(
    _agg_split_body,
    out_type=jax.ShapeDtypeStruct((2, PR, 128), F32),
    mesh=_SC_MESH,
    scratch_types=_SC_SCRATCH,
    compiler_params=_SC_PARAMS,
    name="sc_agg_split",
)


BLK = PR // 8
_GRID = PR // BLK


def _leaky(x):
    return jnp.where(x >= 0, x, 0.01 * x)


def _mm1_body(p_ref, w_ref, b_ref, o_ref):
    a = p_ref[0] + p_ref[1]
    for c in range(4):
        acc = jnp.dot(a, w_ref[c], preferred_element_type=F32)
        o_ref[c] = _leaky(acc + b_ref[c, :][None, :])


def _mm_mid_body(a_ref, w_ref, b_ref, o_ref):
    for c in range(4):
        acc = jnp.dot(a_ref[0], w_ref[0, c], preferred_element_type=F32)
        for kc in range(1, 4):
            acc += jnp.dot(a_ref[kc], w_ref[kc, c],
                           preferred_element_type=F32)
        o_ref[c] = _leaky(acc + b_ref[c, :][None, :])


def _mm_last_body(a_ref, w_ref, o_ref):
    acc = jnp.dot(a_ref[0], w_ref[0], preferred_element_type=F32)
    for kc in range(1, 4):
        acc += jnp.dot(a_ref[kc], w_ref[kc], preferred_element_type=F32)
    o_ref[...] = acc


def _sum_bias_body(p_ref, b_ref, o_ref):
    o_ref[...] = p_ref[0] + p_ref[1] + b_ref[0, :][None, :]


def _full(shape):
    return pl.BlockSpec(shape, lambda i: tuple(0 for _ in shape))


_mm1 = pl.pallas_call(
    _mm1_body,
    grid=(_GRID,),
    in_specs=[pl.BlockSpec((2, BLK, 128), lambda i: (0, i, 0)),
              _full((4, 128, 128)), _full((4, 128))],
    out_specs=pl.BlockSpec((4, BLK, 128), lambda i: (0, i, 0)),
    out_shape=jax.ShapeDtypeStruct((4, PR, 128), F32),
)

_mm_mid = pl.pallas_call(
    _mm_mid_body,
    grid=(_GRID,),
    in_specs=[pl.BlockSpec((4, BLK, 128), lambda i: (0, i, 0)),
              _full((4, 4, 128, 128)), _full((4, 128))],
    out_specs=pl.BlockSpec((4, BLK, 128), lambda i: (0, i, 0)),
    out_shape=jax.ShapeDtypeStruct((4, PR, 128), F32),
)

_mm_last = pl.pallas_call(
    _mm_last_body,
    grid=(_GRID,),
    in_specs=[pl.BlockSpec((4, BLK, 128), lambda i: (0, i, 0)),
              _full((4, 128, 128))],
    out_specs=pl.BlockSpec((BLK, 128), lambda i: (i, 0)),
    out_shape=jax.ShapeDtypeStruct((PR, 128), F32),
)

_sum_bias = pl.pallas_call(
    _sum_bias_body,
    grid=(_GRID,),
    in_specs=[pl.BlockSpec((2, BLK, 128), lambda i: (0, i, 0)),
              _full((1, 128))],
    out_specs=pl.BlockSpec((BLK, 128), lambda i: (i, 0)),
    out_shape=jax.ShapeDtypeStruct((PR, 128), F32),
)


def _bd8(w16):
    w = jnp.zeros((16, 16), F32).at[:w16.shape[0], :w16.shape[1]].set(w16)
    return jnp.kron(jnp.eye(8, dtype=F32), w)


def kernel(features, edge_index, W1, b1, W2, b2, W5, b5):
    src = edge_index[0].astype(I32)
    dst = edge_index[1].astype(I32)
    src3 = src.reshape(NSB, KB, B)
    dst3 = dst.reshape(NSB, KB, B)
    sd_split = jnp.concatenate([src3, dst3], axis=1)
    src4 = src3[None] + (jnp.arange(4, dtype=I32) * NPAD)[:, None, None,
                                                          None]
    dst4 = jnp.broadcast_to(dst3[None], (4, NSB, KB, B))
    sd_mid = jnp.concatenate([src4, dst4], axis=2)

    feat16 = _linear_put(jnp.pad(features, ((0, 0), (0, 12))))

    bd1 = jnp.stack([_bd8(W1[:, 16 * c:16 * (c + 1)]) for c in range(4)])
    bd2 = jnp.stack(
        [jnp.stack([_bd8(W2[16 * kc:16 * (kc + 1), 16 * c:16 * (c + 1)])
                    for c in range(4)]) for kc in range(4)])
    bd5 = jnp.stack([_bd8(W5[16 * kc:16 * (kc + 1), :]) for kc in range(4)])
    b1p = jnp.stack([jnp.tile(b1[16 * c:16 * (c + 1)], 8) for c in range(4)])
    b2p = jnp.stack([jnp.tile(b2[16 * c:16 * (c + 1)], 8) for c in range(4)])
    b5p = jnp.tile(jnp.pad(b5, (0, 13)), 8).reshape(1, 128)

    p = _agg_split(feat16, sd_split)
    x = _mm1(p, bd1, b1p)
    for _ in range(7):
        a = _agg_mid(_as_table(x, 4 * NPAD), sd_mid)
        x = _mm_mid(a, bd2, b2p)
    y = _mm_last(x, bd5)
    p = _agg_split(_as_table(y, NPAD), sd_split)
    o = _sum_bias(p, b5p)
    return o.reshape(NPAD, 16)[:N, :3]


# speedup vs baseline: 15.3543x; 1.0141x over previous
import jax
import jax.numpy as jnp
from jax import lax
from jax.experimental import pallas as pl
from jax.experimental.pallas import tpu as pltpu
from jax.experimental.pallas import tpu_sc as plsc
from jax.experimental.layout import Format, Layout, with_layout_constraint

N = 100000
NPAD = 100352
PR = NPAD // 8
E = 1600000
B = 125
KB = 4
NROWS = E // B
NSB = NROWS // KB
U = 64
NU = NPAD // U
U0 = 64
NU0 = NPAD // U0
F32 = jnp.float32
I32 = jnp.int32

_T16 = Layout(major_to_minor=(0, 1), tiling=((16,),))


def _as_table(x_packed, n_rows):
    t = x_packed.reshape(n_rows, 16)
    return with_layout_constraint(t, _T16)


def _linear_put(x):
    sharding = jax.sharding.SingleDeviceSharding(jax.devices()[0])
    return jax.device_put(x, Format(_T16, sharding))


def _zero_fill(buf, nrows):
    def body(i, _):
        buf[i, :] = jnp.zeros((16,), F32)
        return 0
    lax.fori_loop(0, nrows, body, 0)


def _edge_pass(xflat, sd, worker, nworkers, acc, sdbuf, rows,
               sem_g, sem_s, sem_i):
    T = NSB // nworkers

    def stage_sync(t, slot):
        pltpu.sync_copy(sd.at[worker + nworkers * t], sdbuf.at[slot])

    def stage_fire(t, slot):
        pltpu.async_copy(sd.at[worker + nworkers * t], sdbuf.at[slot],
                         sem_i)

    def stage_wait(t, slot):
        pltpu.make_async_copy(sd.at[worker + nworkers * t], sdbuf.at[slot],
                              sem_i).wait()

    def fire_g(slot):
        for j in range(KB):
            pltpu.async_copy(xflat.at[sdbuf.at[slot].at[j]],
                             rows.at[slot * KB + j], sem_g)

    def drain_g(slot):
        for j in range(KB):
            pltpu.make_async_copy(xflat.at[sdbuf.at[slot].at[j]],
                                  rows.at[slot * KB + j], sem_g).wait()

    def fire_s(slot):
        for j in range(KB):
            pltpu.async_copy(rows.at[slot * KB + j],
                             acc.at[sdbuf.at[slot].at[KB + j]], sem_s,
                             add=True)

    def drain_s(slot):
        for j in range(KB):
            pltpu.make_async_copy(rows.at[slot * KB + j],
                                  acc.at[sdbuf.at[slot].at[KB + j]],
                                  sem_s).wait()

    stage_sync(0, 0)
    stage_sync(1, 1)
    fire_g(0)
    fire_g(1)

    def body(t, _):
        slot = lax.rem(t, 3)
        nslot = lax.rem(t + 2, 3)

        @pl.when(t > 0)
        def _():
            drain_s(nslot)

        @pl.when(t + 2 < T)
        def _():
            stage_fire(t + 2, nslot)

        drain_g(slot)
        fire_s(slot)

        @pl.when(t + 2 < T)
        def _():
            stage_wait(t + 2, nslot)
            fire_g(nslot)
        return 0

    lax.fori_loop(0, T, body, 0)
    drain_s(lax.rem(T - 1, 3))


def _zero_acc(acc, zbuf, sub):
    def body(t, _):
        u = sub + 16 * t
        pltpu.sync_copy(zbuf, acc.at[pl.ds(u * U0, U0)])
        return 0
    lax.fori_loop(0, NU0 // 16, body, 0)


def _writeback(acc, zbuf, stage, stage2, out_plane, sub):
    def body(t, _):
        u = sub + 16 * t
        pltpu.sync_copy(acc.at[pl.ds(u * U, U)], stage)
        pltpu.sync_copy(zbuf.at[pl.ds(0, U)], acc.at[pl.ds(u * U, U)])
        for r8 in range(U // 8):
            for s in range(8):
                stage2[r8, pl.ds(16 * s, 16)] = stage[r8 * 8 + s, :]
        row0 = pl.multiple_of(u * (U // 8), 8)
        pltpu.sync_copy(stage2, out_plane.at[pl.ds(row0, U // 8)])
        return 0
    lax.fori_loop(0, NU // 16, body, 0)


def _agg_mid_body(xflat, sdall, out,
                  acc, sdbuf, rows, zbuf, stage, stage2, sem_g, sem_s,
                  sem_i):
    core = lax.axis_index("c")
    sub = lax.axis_index("s")
    _zero_fill(zbuf, U0)
    _zero_acc(acc, zbuf, sub)
    for l in range(2):
        cc = 2 * core + l
        plsc.subcore_barrier()
        _edge_pass(xflat, sdall.at[cc], sub, 16,
                   acc, sdbuf, rows, sem_g, sem_s, sem_i)
        plsc.subcore_barrier()
        _writeback(acc, zbuf, stage, stage2, out.at[cc], sub)


def _agg_split_body(xtab, sd, out,
                    acc, sdbuf, rows, zbuf, stage, stage2, sem_g, sem_s,
                    sem_i):
    core = lax.axis_index("c")
    sub = lax.axis_index("s")
    _zero_fill(zbuf, U0)
    wid = core * 16 + sub
    _zero_acc(acc, zbuf, sub)
    plsc.subcore_barrier()
    _edge_pass(xtab, sd, wid, 32, acc, sdbuf, rows, sem_g, sem_s,
                sem_i)
    plsc.subcore_barrier()
    _writeback(acc, zbuf, stage, stage2, out.at[core], sub)


_SC_SCRATCH = [
    pltpu.VMEM_SHARED((NPAD, 16), F32),
    pltpu.VMEM((3, 2 * KB, B), I32),
    pltpu.VMEM((3 * KB, B, 16), F32),
    pltpu.VMEM((U0, 16), F32),
    pltpu.VMEM((U, 16), F32),
    pltpu.VMEM((U // 8, 128), F32),
    pltpu.SemaphoreType.DMA,
    pltpu.SemaphoreType.DMA,
    pltpu.SemaphoreType.DMA,
]
_SC_MESH = plsc.VectorSubcoreMesh(core_axis_name="c", subcore_axis_name="s")
_SC_PARAMS = pltpu.CompilerParams(use_tc_tiling_on_sc=False)

_agg_mid = pl.kernel(
    _agg_mid_body,
    out_type=jax.ShapeDtypeStruct((4, PR, 128), F32),
    mesh=_SC_MESH,
    scratch_types=_SC_SCRATCH,
    compiler_params=_SC_PARAMS,
    name="sc_agg_mid",
)

_agg_split = pl.kernel(
    _agg_split_body,
    out_type=jax.ShapeDtypeStruct((2, PR, 128), F32),
    mesh=_SC_MESH,
    scratch_types=_SC_SCRATCH,
    compiler_params=_SC_PARAMS,
    name="sc_agg_split",
)


BLK = PR // 8
_GRID = PR // BLK


def _leaky(x):
    return jnp.where(x >= 0, x, 0.01 * x)


def _mm1_body(p_ref, w_ref, b_ref, o_ref):
    a = p_ref[0] + p_ref[1]
    for c in range(4):
        acc = jnp.dot(a, w_ref[c], preferred_element_type=F32)
        o_ref[c] = _leaky(acc + b_ref[c, :][None, :])


def _mm_mid_body(a_ref, w_ref, b_ref, o_ref):
    for c in range(4):
        acc = jnp.dot(a_ref[0], w_ref[0, c], preferred_element_type=F32)
        for kc in range(1, 4):
            acc += jnp.dot(a_ref[kc], w_ref[kc, c],
                           preferred_element_type=F32)
        o_ref[c] = _leaky(acc + b_ref[c, :][None, :])


def _mm_last_body(a_ref, w_ref, o_ref):
    acc = jnp.dot(a_ref[0], w_ref[0], preferred_element_type=F32)
    for kc in range(1, 4):
        acc += jnp.dot(a_ref[kc], w_ref[kc], preferred_element_type=F32)
    o_ref[...] = acc


def _sum_bias_body(p_ref, b_ref, o_ref):
    o_ref[...] = p_ref[0] + p_ref[1] + b_ref[0, :][None, :]


def _full(shape):
    return pl.BlockSpec(shape, lambda i: tuple(0 for _ in shape))


_mm1 = pl.pallas_call(
    _mm1_body,
    grid=(_GRID,),
    in_specs=[pl.BlockSpec((2, BLK, 128), lambda i: (0, i, 0)),
              _full((4, 128, 128)), _full((4, 128))],
    out_specs=pl.BlockSpec((4, BLK, 128), lambda i: (0, i, 0)),
    out_shape=jax.ShapeDtypeStruct((4, PR, 128), F32),
)

_mm_mid = pl.pallas_call(
    _mm_mid_body,
    grid=(_GRID,),
    in_specs=[pl.BlockSpec((4, BLK, 128), lambda i: (0, i, 0)),
              _full((4, 4, 128, 128)), _full((4, 128))],
    out_specs=pl.BlockSpec((4, BLK, 128), lambda i: (0, i, 0)),
    out_shape=jax.ShapeDtypeStruct((4, PR, 128), F32),
)

_mm_last = pl.pallas_call(
    _mm_last_body,
    grid=(_GRID,),
    in_specs=[pl.BlockSpec((4, BLK, 128), lambda i: (0, i, 0)),
              _full((4, 128, 128))],
    out_specs=pl.BlockSpec((BLK, 128), lambda i: (i, 0)),
    out_shape=jax.ShapeDtypeStruct((PR, 128), F32),
)

_sum_bias = pl.pallas_call(
    _sum_bias_body,
    grid=(_GRID,),
    in_specs=[pl.BlockSpec((2, BLK, 128), lambda i: (0, i, 0)),
              _full((1, 128))],
    out_specs=pl.BlockSpec((BLK, 128), lambda i: (i, 0)),
    out_shape=jax.ShapeDtypeStruct((PR, 128), F32),
)


def _bd8(w16):
    w = jnp.zeros((16, 16), F32).at[:w16.shape[0], :w16.shape[1]].set(w16)
    return jnp.kron(jnp.eye(8, dtype=F32), w)


def kernel(features, edge_index, W1, b1, W2, b2, W5, b5):
    src = edge_index[0].astype(I32)
    dst = edge_index[1].astype(I32)
    src3 = src.reshape(NSB, KB, B)
    dst3 = dst.reshape(NSB, KB, B)
    sd_split = jnp.concatenate([src3, dst3], axis=1)
    src4 = src3[None] + (jnp.arange(4, dtype=I32) * NPAD)[:, None, None,
                                                          None]
    dst4 = jnp.broadcast_to(dst3[None], (4, NSB, KB, B))
    sd_mid = jnp.concatenate([src4, dst4], axis=2)

    feat16 = _linear_put(jnp.pad(features, ((0, 0), (0, 12))))

    bd1 = jnp.stack([_bd8(W1[:, 16 * c:16 * (c + 1)]) for c in range(4)])
    bd2 = jnp.stack(
        [jnp.stack([_bd8(W2[16 * kc:16 * (kc + 1), 16 * c:16 * (c + 1)])
                    for c in range(4)]) for kc in range(4)])
    bd5 = jnp.stack([_bd8(W5[16 * kc:16 * (kc + 1), :]) for kc in range(4)])
    b1p = jnp.stack([jnp.tile(b1[16 * c:16 * (c + 1)], 8) for c in range(4)])
    b2p = jnp.stack([jnp.tile(b2[16 * c:16 * (c + 1)], 8) for c in range(4)])
    b5p = jnp.tile(jnp.pad(b5, (0, 13)), 8).reshape(1, 128)

    p = _agg_split(feat16, sd_split)
    x = _mm1(p, bd1, b1p)
    for _ in range(7):
        a = _agg_mid(_as_table(x, 4 * NPAD), sd_mid)
        x = _mm_mid(a, bd2, b2p)
    y = _mm_last(x, bd5)
    p = _agg_split(_as_table(y, NPAD), sd_split)
    o = _sum_bias(p, b5p)
    return o.reshape(NPAD, 16)[:N, :3]
